# baseline (speedup 1.0000x reference)
import jax
import jax.numpy as jnp
from jax.experimental import pallas as pl
from jax.experimental.pallas import tpu as pltpu

H = 50
W = 50
C = 512
PW = 52
M = H * PW
NPIX = H * W
OFFS = tuple((t // 3) * PW + (t % 3) for t in range(9))
PCH = NPIX // 5


def _conv_kernel(x_ref, w_ref, wreg_ref, wcls_ref, bsw_ref, breg_ref,
                 bcls_ref, hc_ref, xscr):
    xscr[:, :] = jnp.zeros((C, PW * 53), jnp.bfloat16)
    xb = x_ref[:, :].astype(jnp.bfloat16)
    for h in range(H):
        xscr[:, (h + 1) * PW + 1:(h + 1) * PW + 1 + W] = \
            xb[:, h * W:(h + 1) * W]

    acc = jnp.zeros((C, M), jnp.float32)
    for t in range(9):
        acc = acc + jax.lax.dot_general(
            w_ref[t], xscr[:, OFFS[t]:OFFS[t] + M],
            (((1,), (0,)), ((), ())),
            preferred_element_type=jnp.float32)

    bsw = jnp.transpose(bsw_ref[:, :])
    feats = jnp.maximum(acc + bsw, 0.0).astype(jnp.bfloat16)
    wh = jnp.concatenate(
        [wreg_ref[:, :], wcls_ref[:, :]], axis=0).astype(jnp.bfloat16)
    bh = jnp.transpose(
        jnp.concatenate([breg_ref[:, :], bcls_ref[:, :]], axis=1))
    head = jax.lax.dot_general(
        wh, feats, (((1,), (0,)), ((), ())),
        preferred_element_type=jnp.float32) + bh
    hT = jnp.transpose(head)
    hc_ref[:, :] = jnp.concatenate(
        [hT[h * PW:h * PW + W, :] for h in range(H)], axis=0)


def _inter_kernel(hc_ref, reg_ref, cls_ref):
    c = pl.program_id(0)
    hs = hc_ref[pl.ds(c * PCH, PCH), :]
    rep = jnp.broadcast_to(
        hs[:, None, :], (PCH, 9, 54)).reshape(PCH * 9, 54)
    a = jax.lax.broadcasted_iota(jnp.int32, (PCH * 9, 54), 0) % 9
    y = rep
    for k in range(4):
        y = jnp.where((a >> k) & 1 == 1,
                      jnp.roll(y, -(4 << k), axis=1), y)
    zc = jnp.roll(rep, -36, axis=1)
    for k in range(4):
        zc = jnp.where((a >> k) & 1 == 1,
                       jnp.roll(zc, -(2 << k), axis=1), zc)
    reg_ref[pl.ds(c * PCH * 9, PCH * 9), :] = y[:, :4]
    cls_ref[pl.ds(c * PCH * 9, PCH * 9), :] = zc[:, :2]


def kernel(x, W_sw, b_sw, W_cls, b_cls, W_reg, b_reg):
    xf = x.reshape(C, NPIX)
    w9 = jnp.transpose(
        W_sw.astype(jnp.bfloat16).reshape(C, C, 9), (2, 0, 1))
    wreg = W_reg.reshape(36, C)
    wcls = W_cls.reshape(18, C)
    bsw = b_sw.reshape(1, C)
    breg = b_reg.reshape(1, 36)
    bcls = b_cls.reshape(1, 18)

    hc = pl.pallas_call(
        _conv_kernel,
        out_shape=jax.ShapeDtypeStruct((NPIX, 54), jnp.float32),
        scratch_shapes=[pltpu.VMEM((C, PW * 53), jnp.bfloat16)],
    )(xf, w9, wreg, wcls, bsw, breg, bcls)

    reg, cls = pl.pallas_call(
        _inter_kernel,
        grid=(5,),
        in_specs=[pl.BlockSpec((NPIX, 54), lambda c: (0, 0))],
        out_specs=(pl.BlockSpec((NPIX * 9, 4), lambda c: (0, 0)),
                   pl.BlockSpec((NPIX * 9, 2), lambda c: (0, 0))),
        out_shape=(jax.ShapeDtypeStruct((NPIX * 9, 4), jnp.float32),
                   jax.ShapeDtypeStruct((NPIX * 9, 2), jnp.float32)),
    )(hc)
    return (reg.reshape(1, NPIX * 9, 4), cls.reshape(1, NPIX * 9, 2))


# speedup vs baseline: 1.1096x; 1.1096x over previous
import jax
import jax.numpy as jnp
from jax.experimental import pallas as pl
from jax.experimental.pallas import tpu as pltpu

H = 50
W = 50
C = 512
PW = 52
M = H * PW
NPIX = H * W
OFFS = tuple((t // 3) * PW + (t % 3) for t in range(9))
PCH = NPIX // 5


def _conv_kernel(x_ref, w_ref, wreg_ref, wcls_ref, bsw_ref, breg_ref,
                 bcls_ref, hc_ref, xscr):
    xscr[:, :] = jnp.zeros((C, PW * 53), jnp.bfloat16)
    for h in range(H):
        xscr[:, (h + 1) * PW + 1:(h + 1) * PW + 1 + W] = \
            x_ref[:, h, :].astype(jnp.bfloat16)

    acc = jnp.zeros((C, M), jnp.float32)
    for t in range(9):
        acc = acc + jax.lax.dot_general(
            w_ref[t], xscr[:, OFFS[t]:OFFS[t] + M],
            (((1,), (0,)), ((), ())),
            preferred_element_type=jnp.float32)

    bsw = jnp.transpose(bsw_ref[:, :])
    feats = jnp.maximum(acc + bsw, 0.0).astype(jnp.bfloat16)
    wh = jnp.concatenate(
        [wreg_ref[:, :], wcls_ref[:, :],
         jnp.zeros((128 - 54, C), jnp.bfloat16)],
        axis=0).astype(jnp.bfloat16)
    bh = jnp.transpose(
        jnp.concatenate([breg_ref[:, :], bcls_ref[:, :],
                         jnp.zeros((1, 128 - 54), jnp.float32)], axis=1))
    head = jax.lax.dot_general(
        wh, feats, (((1,), (0,)), ((), ())),
        preferred_element_type=jnp.float32) + bh
    hT = jnp.transpose(head)
    hc_ref[:, :] = jnp.concatenate(
        [hT[h * PW:h * PW + W, :] for h in range(H)], axis=0)


def _inter_kernel(hc_ref, reg_ref, cls_ref):
    c = pl.program_id(0)
    hs = hc_ref[pl.ds(c * PCH, PCH), :]
    rep = jnp.broadcast_to(
        hs[:, None, :], (PCH, 9, 128)).reshape(PCH * 9, 128)
    a = jax.lax.broadcasted_iota(jnp.int32, (PCH * 9, 128), 0) % 9
    y = rep
    for k in range(4):
        y = jnp.where((a >> k) & 1 == 1,
                      jnp.roll(y, -(4 << k), axis=1), y)
    zc = jnp.roll(rep, -36, axis=1)
    for k in range(4):
        zc = jnp.where((a >> k) & 1 == 1,
                       jnp.roll(zc, -(2 << k), axis=1), zc)
    reg_ref[pl.ds(c * PCH * 9, PCH * 9), :] = y[:, :4]
    cls_ref[pl.ds(c * PCH * 9, PCH * 9), :] = zc[:, :2]


def kernel(x, W_sw, b_sw, W_cls, b_cls, W_reg, b_reg):
    xf = x.reshape(C, H, W)
    w9 = jnp.transpose(
        W_sw.astype(jnp.bfloat16).reshape(C, C, 9), (2, 0, 1))
    wreg = W_reg.reshape(36, C)
    wcls = W_cls.reshape(18, C)
    bsw = b_sw.reshape(1, C)
    breg = b_reg.reshape(1, 36)
    bcls = b_cls.reshape(1, 18)

    hc = pl.pallas_call(
        _conv_kernel,
        out_shape=jax.ShapeDtypeStruct((NPIX, 128), jnp.float32),
        scratch_shapes=[pltpu.VMEM((C, PW * 53), jnp.bfloat16)],
    )(xf, w9, wreg, wcls, bsw, breg, bcls)

    reg, cls = pl.pallas_call(
        _inter_kernel,
        grid=(5,),
        in_specs=[pl.BlockSpec((NPIX, 128), lambda c: (0, 0))],
        out_specs=(pl.BlockSpec((NPIX * 9, 4), lambda c: (0, 0)),
                   pl.BlockSpec((NPIX * 9, 2), lambda c: (0, 0))),
        out_shape=(jax.ShapeDtypeStruct((NPIX * 9, 4), jnp.float32),
                   jax.ShapeDtypeStruct((NPIX * 9, 2), jnp.float32)),
    )(hc)
    return (reg.reshape(1, NPIX * 9, 4), cls.reshape(1, NPIX * 9, 2))


# speedup vs baseline: 1.7683x; 1.5936x over previous
import jax
import jax.numpy as jnp
from jax.experimental import pallas as pl

H = 50
W = 50
C = 512
PW = W + 2
M = H * PW
NPIX = H * W
NA = 9


def _rpn_kernel(x_ref, w_ref, wreg_ref, wcls_ref, bsw_ref, breg_ref,
                bcls_ref, reg_ref, cls_ref):
    acc = jnp.zeros((C, M), dtype=jnp.float32)
    for t in range(NA):
        s = (t // 3) * PW + (t % 3)
        acc = acc + jax.lax.dot_general(
            w_ref[t], x_ref[:, s:s + M],
            (((0,), (0,)), ((), ())),
            preferred_element_type=jnp.float32)
    bsw = jnp.transpose(bsw_ref[:, :])
    feats = jnp.maximum(acc + bsw, 0.0).astype(jnp.bfloat16)

    wh = jnp.concatenate(
        [wreg_ref[:, :], wcls_ref[:, :]], axis=0).astype(jnp.bfloat16)
    bh = jnp.transpose(
        jnp.concatenate([breg_ref[:, :], bcls_ref[:, :]], axis=1))
    head = jax.lax.dot_general(
        wh, feats, (((1,), (0,)), ((), ())),
        preferred_element_type=jnp.float32) + bh

    hT = jnp.transpose(head)
    hC = jnp.concatenate(
        [hT[h * PW:h * PW + W, :] for h in range(H)], axis=0)
    reg_ref[:, :] = hC[:, :36]
    cls_ref[:, :] = hC[:, 36:54]


def kernel(x, W_sw, b_sw, W_cls, b_cls, W_reg, b_reg):
    xr = x.reshape(C, H, W).astype(jnp.bfloat16)
    xpad = jax.lax.dynamic_update_slice(
        jnp.zeros((C, H + 3, PW), jnp.bfloat16), xr, (0, 1, 1))
    xflat = xpad.reshape(C, (H + 3) * PW)

    wflat = jnp.transpose(
        W_sw.astype(jnp.bfloat16).reshape(C, C, NA), (2, 1, 0))
    wreg = W_reg.reshape(36, C)
    wcls = W_cls.reshape(18, C)
    bsw = b_sw.reshape(1, C)
    breg = b_reg.reshape(1, 36)
    bcls = b_cls.reshape(1, 18)

    reg, cls = pl.pallas_call(
        _rpn_kernel,
        out_shape=(jax.ShapeDtypeStruct((NPIX, 36), jnp.float32),
                   jax.ShapeDtypeStruct((NPIX, 18), jnp.float32)),
    )(xflat, wflat, wreg, wcls, bsw, breg, bcls)
    return (reg.reshape(1, NPIX * NA, 4), cls.reshape(1, NPIX * NA, 2))


# speedup vs baseline: 1.8335x; 1.0369x over previous
import jax
import jax.numpy as jnp
from jax.experimental import pallas as pl

H = 50
W = 50
C = 512
PW = W + 2
M = H * PW
NPIX = H * W
NA = 9


def _rpn_kernel(x_ref, w_ref, wreg_ref, wcls_ref, bsw_ref, breg_ref,
                bcls_ref, reg_ref, cls_ref):
    acc = jnp.zeros((C, M), dtype=jnp.float32)
    for t in range(NA):
        s = (t // 3) * PW + (t % 3)
        acc = acc + jax.lax.dot_general(
            w_ref[t], x_ref[:, s:s + M],
            (((1,), (0,)), ((), ())),
            preferred_element_type=jnp.float32)
    bsw = jnp.transpose(bsw_ref[:, :])
    feats = jnp.maximum(acc + bsw, 0.0).astype(jnp.bfloat16)

    wh = jnp.concatenate(
        [wreg_ref[:, :], wcls_ref[:, :]], axis=0).astype(jnp.bfloat16)
    bh = jnp.transpose(
        jnp.concatenate([breg_ref[:, :], bcls_ref[:, :]], axis=1))
    head = jax.lax.dot_general(
        wh, feats, (((1,), (0,)), ((), ())),
        preferred_element_type=jnp.float32) + bh

    hT = jnp.transpose(head)
    hC = jnp.concatenate(
        [hT[h * PW:h * PW + W, :] for h in range(H)], axis=0)
    reg_ref[:, :] = hC[:, :36]
    cls_ref[:, :] = hC[:, 36:54]


def kernel(x, W_sw, b_sw, W_cls, b_cls, W_reg, b_reg):
    xr = x.reshape(C, H, W).astype(jnp.bfloat16)
    xpad = jax.lax.dynamic_update_slice(
        jnp.zeros((C, H + 3, PW), jnp.bfloat16), xr, (0, 1, 1))
    xflat = xpad.reshape(C, (H + 3) * PW)

    wflat = jnp.transpose(
        W_sw.reshape(C, C, NA), (2, 0, 1)).astype(jnp.bfloat16)
    wreg = W_reg.reshape(36, C)
    wcls = W_cls.reshape(18, C)
    bsw = b_sw.reshape(1, C)
    breg = b_reg.reshape(1, 36)
    bcls = b_cls.reshape(1, 18)

    reg, cls = pl.pallas_call(
        _rpn_kernel,
        out_shape=(jax.ShapeDtypeStruct((NPIX, 36), jnp.float32),
                   jax.ShapeDtypeStruct((NPIX, 18), jnp.float32)),
    )(xflat, wflat, wreg, wcls, bsw, breg, bcls)
    return (reg.reshape(1, NPIX * NA, 4), cls.reshape(1, NPIX * NA, 2))
